# baseline (speedup 1.0000x reference)
import functools

import jax
import jax.numpy as jnp
from jax import lax
from jax.experimental import pallas as pl
from jax.experimental.pallas import tpu as pltpu
from jax.experimental.pallas import tpu_sc as plsc

B = 16384
D = 64
L = 16
NC = 2
NS = 16
NW = NC * NS
BPW = B // NW
PH = 128
NPH = BPW // PH
NGROUP = PH // L

_mesh = plsc.VectorSubcoreMesh(core_axis_name="c", subcore_axis_name="s")


@functools.partial(
    pl.kernel,
    mesh=_mesh,
    compiler_params=pltpu.CompilerParams(needs_layout_passes=False),
    out_type=jax.ShapeDtypeStruct((B,), jnp.float32),
    scratch_types=[
        pltpu.VMEM((BPW,), jnp.int32),
        pltpu.VMEM((BPW,), jnp.int32),
        pltpu.VMEM((2, PH, D), jnp.float32),
        pltpu.VMEM((2, PH, D), jnp.float32),
        pltpu.VMEM((D,), jnp.float32),
        pltpu.VMEM((L,), jnp.float32),
        pltpu.VMEM((BPW,), jnp.float32),
        pltpu.SemaphoreType.DMA,
        pltpu.SemaphoreType.DMA,
    ],
)
def _sc_kernel(uid_hbm, iid_hbm, ut_hbm, it_hbm, w_hbm, b_hbm, out_hbm,
               uidx_v, iidx_v, urows_v, irows_v, w_v, b_v, out_v,
               usem, isem):
    wid = lax.axis_index("s") * NC + lax.axis_index("c")
    base = wid * BPW

    pltpu.sync_copy(uid_hbm.at[pl.ds(base, BPW)], uidx_v)
    pltpu.sync_copy(iid_hbm.at[pl.ds(base, BPW)], iidx_v)
    pltpu.sync_copy(w_hbm, w_v)
    pltpu.sync_copy(b_hbm, b_v)

    wvecs = [w_v[pl.ds(j * L, L)] for j in range(D // L)]
    bvec = b_v[...]
    liota = lax.iota(jnp.int32, L)

    def fire(p, buf):
        copies = []
        pbase = p * PH
        for k in range(PH):
            if k % L == 0:
                uvec = uidx_v[pl.ds(pbase + k, L)]
                ivec = iidx_v[pl.ds(pbase + k, L)]
            u = uvec[k % L]
            i = ivec[k % L]
            copies.append(pltpu.async_copy(
                ut_hbm.at[u], urows_v.at[buf, k], usem))
            copies.append(pltpu.async_copy(
                it_hbm.at[i], irows_v.at[buf, k], isem))
        return copies

    def compute(p, buf):
        pbase = p * PH

        def group_body(g, carry):
            r0 = g * L
            acc = bvec
            for rr in range(L):
                r = r0 + rr
                s = None
                for j in range(D // L):
                    t = (urows_v[buf, r, pl.ds(j * L, L)]
                         * irows_v[buf, r, pl.ds(j * L, L)] * wvecs[j])
                    s = t if s is None else s + t
                acc = jnp.where(liota == rr, acc + jnp.sum(s), acc)
            out_v[pl.ds(pbase + r0, L)] = acc
            return carry

        lax.fori_loop(0, NGROUP, group_body, 0, unroll=False)

    inflight = fire(0, 0)
    for p in range(NPH):
        nxt = fire(p + 1, (p + 1) % 2) if p + 1 < NPH else []
        for cp in inflight:
            cp.wait()
        compute(p, p % 2)
        inflight = nxt

    pltpu.sync_copy(out_v, out_hbm.at[pl.ds(base, BPW)])


def kernel(user_ids, item_ids, user_table, item_table, fc_w, fc_b):
    w = fc_w.reshape(D)
    b = jnp.broadcast_to(fc_b.reshape(1), (L,))
    return _sc_kernel(user_ids, item_ids, user_table, item_table, w, b)
